# baseline (speedup 1.0000x reference)
import functools

import numpy as np
import jax
import jax.numpy as jnp
from jax import lax
from jax.experimental import pallas as pl
from jax.experimental.pallas import tpu as pltpu
from jax.experimental.pallas import tpu_sc as plsc
from jax.sharding import Mesh, PartitionSpec as P

try:
    from jax.experimental.shard_map import shard_map
except ImportError:
    shard_map = jax.shard_map

N_CODES = 8192
HALF = N_CODES // 2
DIM = 64
DEPTH = 4
ROWS = 8192
TILE_R = 1024
NW = 32

_HI = jax.lax.Precision.HIGHEST
_DN = (((1,), (1,)), ((), ()))


def _first_argmin(dist, col):
    m = jnp.min(dist, axis=1, keepdims=True)
    idx = jnp.min(jnp.where(dist == m, col, N_CODES), axis=1, keepdims=True)
    return m, idx


def _choose(r, cbb_ref, cbn_ref, windowed):
    s = jax.lax.dot_general(r.astype(jnp.bfloat16), cbb_ref[...], _DN,
                            preferred_element_type=jnp.float32)
    in_n = jnp.sum(r * r, axis=1, keepdims=True)
    dist = (in_n + cbn_ref[...]) - 2.0 * s
    if windowed:
        d0 = dist[:, :HALF]
        d1 = dist[:, HALF:]
        m0 = jnp.min(d0, axis=1, keepdims=True)
        i0 = jnp.argmin(d0, axis=1).astype(jnp.int32)[:, None]
        m1 = jnp.min(d1, axis=1, keepdims=True)
        i1 = jnp.argmin(d1, axis=1).astype(jnp.int32)[:, None] + HALF
        take1 = m1 < m0.astype(jnp.bfloat16).astype(jnp.float32)
        return jnp.where(take1, i1, i0)
    return jnp.argmin(dist, axis=1).astype(jnp.int32)[:, None]


def _depth0_body(x_ref, cbb_ref, cbn_ref, idx_ref):
    idx_ref[...] = _choose(x_ref[...], cbb_ref, cbn_ref, True)


def _depth_body(windowed, rprev_ref, qprev_ref, cbb_ref, cbn_ref,
                idx_ref, r_ref, loss_ref):
    r = rprev_ref[...] - qprev_ref[:, :DIM]
    r_ref[...] = r
    loss_ref[0] = jnp.sum(jnp.sum(r * r, axis=1, keepdims=True),
                          axis=0, keepdims=True)
    idx_ref[...] = _choose(r, cbb_ref, cbn_ref, windowed)


def _final_body(x_ref, rprev_ref, qprev_ref, out_ref, loss_ref):
    r = rprev_ref[...] - qprev_ref[:, :DIM]
    loss_ref[0] = jnp.sum(jnp.sum(r * r, axis=1, keepdims=True),
                          axis=0, keepdims=True)
    x = x_ref[...]
    agg = x - r
    out_ref[...] = x + (agg - x)


def _norms_body(c0, c1, c2, c3, out_ref):
    ones_row = jnp.ones((1, DIM), jnp.float32)
    for d, cr in enumerate((c0, c1, c2, c3)):
        cb = cr[...]
        out_ref[d:d + 1, :] = jax.lax.dot_general(
            ones_row, cb * cb, _DN, precision=_HI)


_row_spec = pl.BlockSpec((TILE_R, DIM), lambda i: (i, 0))
_q_spec = pl.BlockSpec((TILE_R, 2 * DIM), lambda i: (i, 0))
_idx_spec = pl.BlockSpec((TILE_R, 1), lambda i: (i, 0))
_loss_spec = pl.BlockSpec((1, 1, 1), lambda i: (i, 0, 0))
_cbb_spec = pl.BlockSpec((N_CODES, DIM), lambda i: (0, 0))
_cbn_spec = pl.BlockSpec((1, N_CODES), lambda i: (0, 0))
_PAR = pltpu.CompilerParams(dimension_semantics=("parallel",))


def _sc_gather(table, idx, rows):
    b_per_w = rows // NW
    mesh = plsc.VectorSubcoreMesh(core_axis_name="c", subcore_axis_name="s")

    @functools.partial(
        pl.kernel, mesh=mesh,
        out_type=jax.ShapeDtypeStruct((rows, 2 * DIM), jnp.float32),
        scratch_types=[
            pltpu.VMEM((b_per_w,), jnp.int32),
            pltpu.VMEM((b_per_w, 2 * DIM), jnp.float32),
            pltpu.SemaphoreType.DMA,
        ],
    )
    def k(table_hbm, idx_hbm, out_hbm, idx_v, rows_v, sem):
        wid = lax.axis_index("s") * 2 + lax.axis_index("c")
        base = wid * b_per_w
        pltpu.sync_copy(idx_hbm.at[pl.ds(base, b_per_w)], idx_v)
        pltpu.async_copy(table_hbm.at[idx_v], rows_v, sem).wait()
        pltpu.sync_copy(rows_v, out_hbm.at[pl.ds(base, b_per_w)])

    return k(table, idx)


def _pipeline(xf, t0, t1, t2, t3, b0, b1, b2, b3, c0, c1, c2, c3):
    rows = xf.shape[0]
    nt = rows // TILE_R
    tables = (t0, t1, t2, t3)
    cbbs = (b0, b1, b2, b3)
    cbn_all = pl.pallas_call(
        _norms_body,
        in_specs=[pl.BlockSpec((N_CODES, DIM), lambda: (0, 0))] * DEPTH,
        out_specs=pl.BlockSpec((DEPTH, N_CODES), lambda: (0, 0)),
        out_shape=jax.ShapeDtypeStruct((DEPTH, N_CODES), jnp.float32),
    )(c0, c1, c2, c3)
    cbns = [jax.lax.slice(cbn_all, (d, 0), (d + 1, N_CODES))
            for d in range(DEPTH)]
    idx_shape = jax.ShapeDtypeStruct((rows, 1), jnp.int32)
    row_shape = jax.ShapeDtypeStruct((rows, DIM), jnp.float32)
    loss_shape = jax.ShapeDtypeStruct((nt, 1, 1), jnp.float32)

    idx0 = pl.pallas_call(
        _depth0_body,
        grid=(nt,),
        in_specs=[_row_spec, _cbb_spec, _cbn_spec],
        out_specs=_idx_spec,
        out_shape=idx_shape,
        compiler_params=_PAR,
    )(xf, cbbs[0], cbns[0])
    q = _sc_gather(tables[0], idx0.reshape(rows), rows)

    rprev = xf
    idxs = [idx0]
    loss_sums = []
    for d in (1, 2, 3):
        idx_d, rprev, ls = pl.pallas_call(
            functools.partial(_depth_body, d in (0, 3)),
            grid=(nt,),
            in_specs=[_row_spec, _q_spec, _cbb_spec, _cbn_spec],
            out_specs=[_idx_spec, _row_spec, _loss_spec],
            out_shape=[idx_shape, row_shape, loss_shape],
            compiler_params=_PAR,
        )(rprev, q, cbbs[d], cbns[d])
        q = _sc_gather(tables[d], idx_d.reshape(rows), rows)
        idxs.append(idx_d)
        loss_sums.append(ls)

    quants, ls3 = pl.pallas_call(
        _final_body,
        grid=(nt,),
        in_specs=[_row_spec, _row_spec, _q_spec],
        out_specs=[_row_spec, _loss_spec],
        out_shape=[row_shape, loss_shape],
        compiler_params=_PAR,
    )(xf, rprev, q)
    loss_sums.append(ls3)

    codes = jnp.concatenate(idxs, axis=1)
    sums = jnp.stack([jnp.sum(l) for l in loss_sums])[None, :]
    return quants, codes, sums


def kernel(x, C0, C1, C2, C3):
    xf = x.reshape(ROWS, DIM)
    tables = [jnp.pad(C, ((0, 0), (0, DIM))) for C in (C0, C1, C2, C3)]
    cbs = [C[:-1] for C in (C0, C1, C2, C3)]
    cbbs = [c.astype(jnp.bfloat16) for c in cbs]

    quants, codes, sums = _pipeline(xf, *tables, *cbbs, *cbs)

    quants = quants.reshape(x.shape)
    codes = codes.reshape(x.shape[:-1] + (DEPTH,))
    loss = jnp.mean(jnp.sum(sums, axis=0) / (ROWS * DIM))
    return quants, loss, codes


# speedup vs baseline: 1.0166x; 1.0166x over previous
import functools

import numpy as np
import jax
import jax.numpy as jnp
from jax import lax
from jax.experimental import pallas as pl
from jax.experimental.pallas import tpu as pltpu
from jax.experimental.pallas import tpu_sc as plsc
from jax.sharding import Mesh, PartitionSpec as P

try:
    from jax.experimental.shard_map import shard_map
except ImportError:
    shard_map = jax.shard_map

N_CODES = 8192
HALF = N_CODES // 2
DIM = 64
DEPTH = 4
ROWS = 8192
TILE_R = 1024
NW = 32

_HI = jax.lax.Precision.HIGHEST
_DN = (((1,), (1,)), ((), ()))


def _first_argmin(dist, col):
    m = jnp.min(dist, axis=1, keepdims=True)
    idx = jnp.min(jnp.where(dist == m, col, N_CODES), axis=1, keepdims=True)
    return m, idx


def _choose(r, cbb_ref, cbn_ref, windowed):
    s = jax.lax.dot_general(r.astype(jnp.bfloat16), cbb_ref[...], _DN,
                            preferred_element_type=jnp.float32)
    in_n = jnp.sum(r * r, axis=1, keepdims=True)
    dist = (in_n + cbn_ref[...]) - 2.0 * s
    if windowed:
        col = jax.lax.broadcasted_iota(jnp.int32, (TILE_R, HALF), 1)
        m0, i0 = _first_argmin(dist[:, :HALF], col)
        m1, i1 = _first_argmin(dist[:, HALF:], col)
        i1 = i1 + HALF
        take1 = m1 < m0.astype(jnp.bfloat16).astype(jnp.float32)
        return jnp.where(take1, i1, i0), in_n
    return jnp.argmin(dist, axis=1).astype(jnp.int32)[:, None], in_n


def _depth0_body(x_ref, cbb_ref, cbn_ref, idx_ref):
    idx_ref[...] = _choose(x_ref[...], cbb_ref, cbn_ref, True)[0]


def _depth_body(windowed, rprev_ref, qprev_ref, cbb_ref, cbn_ref,
                idx_ref, r_ref, loss_ref):
    r = rprev_ref[...] - qprev_ref[:, :DIM]
    r_ref[...] = r
    idx, in_n = _choose(r, cbb_ref, cbn_ref, windowed)
    idx_ref[...] = idx
    loss_ref[0] = jnp.sum(in_n, axis=0, keepdims=True)


def _final_body(x_ref, rprev_ref, qprev_ref, out_ref, loss_ref):
    r = rprev_ref[...] - qprev_ref[:, :DIM]
    loss_ref[0] = jnp.sum(jnp.sum(r * r, axis=1, keepdims=True),
                          axis=0, keepdims=True)
    x = x_ref[...]
    agg = x - r
    out_ref[...] = x + (agg - x)


def _norms_body(c0, c1, c2, c3, out_ref):
    ones_row = jnp.ones((1, DIM), jnp.float32)
    for d, cr in enumerate((c0, c1, c2, c3)):
        cb = cr[...]
        out_ref[d:d + 1, :] = jax.lax.dot_general(
            ones_row, cb * cb, _DN, precision=_HI)


_row_spec = pl.BlockSpec((TILE_R, DIM), lambda i: (i, 0))
_q_spec = pl.BlockSpec((TILE_R, 2 * DIM), lambda i: (i, 0))
_idx_spec = pl.BlockSpec((TILE_R, 1), lambda i: (i, 0))
_loss_spec = pl.BlockSpec((1, 1, 1), lambda i: (i, 0, 0))
_cbb_spec = pl.BlockSpec((N_CODES, DIM), lambda i: (0, 0))
_cbn_spec = pl.BlockSpec((1, N_CODES), lambda i: (0, 0))
_PAR = pltpu.CompilerParams(dimension_semantics=("parallel",))


def _sc_gather(table, idx, rows):
    b_per_w = rows // NW
    mesh = plsc.VectorSubcoreMesh(core_axis_name="c", subcore_axis_name="s")

    @functools.partial(
        pl.kernel, mesh=mesh,
        out_type=jax.ShapeDtypeStruct((rows, 2 * DIM), jnp.float32),
        scratch_types=[
            pltpu.VMEM((b_per_w,), jnp.int32),
            pltpu.VMEM((b_per_w, 2 * DIM), jnp.float32),
            pltpu.SemaphoreType.DMA,
        ],
    )
    def k(table_hbm, idx_hbm, out_hbm, idx_v, rows_v, sem):
        wid = lax.axis_index("s") * 2 + lax.axis_index("c")
        base = wid * b_per_w
        pltpu.sync_copy(idx_hbm.at[pl.ds(base, b_per_w)], idx_v)
        pltpu.async_copy(table_hbm.at[idx_v], rows_v, sem).wait()
        pltpu.sync_copy(rows_v, out_hbm.at[pl.ds(base, b_per_w)])

    return k(table, idx)


def _pipeline(xf, t0, t1, t2, t3, b0, b1, b2, b3, c0, c1, c2, c3):
    rows = xf.shape[0]
    nt = rows // TILE_R
    tables = (t0, t1, t2, t3)
    cbbs = (b0, b1, b2, b3)
    cbn_all = pl.pallas_call(
        _norms_body,
        in_specs=[pl.BlockSpec((N_CODES, DIM), lambda: (0, 0))] * DEPTH,
        out_specs=pl.BlockSpec((DEPTH, N_CODES), lambda: (0, 0)),
        out_shape=jax.ShapeDtypeStruct((DEPTH, N_CODES), jnp.float32),
    )(c0, c1, c2, c3)
    cbns = [jax.lax.slice(cbn_all, (d, 0), (d + 1, N_CODES))
            for d in range(DEPTH)]
    idx_shape = jax.ShapeDtypeStruct((rows, 1), jnp.int32)
    row_shape = jax.ShapeDtypeStruct((rows, DIM), jnp.float32)
    loss_shape = jax.ShapeDtypeStruct((nt, 1, 1), jnp.float32)

    idx0 = pl.pallas_call(
        _depth0_body,
        grid=(nt,),
        in_specs=[_row_spec, _cbb_spec, _cbn_spec],
        out_specs=_idx_spec,
        out_shape=idx_shape,
        compiler_params=_PAR,
    )(xf, cbbs[0], cbns[0])
    q = _sc_gather(tables[0], idx0.reshape(rows), rows)

    rprev = xf
    idxs = [idx0]
    loss_sums = []
    for d in (1, 2, 3):
        idx_d, rprev, ls = pl.pallas_call(
            functools.partial(_depth_body, d in (0, 3)),
            grid=(nt,),
            in_specs=[_row_spec, _q_spec, _cbb_spec, _cbn_spec],
            out_specs=[_idx_spec, _row_spec, _loss_spec],
            out_shape=[idx_shape, row_shape, loss_shape],
            compiler_params=_PAR,
        )(rprev, q, cbbs[d], cbns[d])
        q = _sc_gather(tables[d], idx_d.reshape(rows), rows)
        idxs.append(idx_d)
        loss_sums.append(ls)

    quants, ls3 = pl.pallas_call(
        _final_body,
        grid=(nt,),
        in_specs=[_row_spec, _row_spec, _q_spec],
        out_specs=[_row_spec, _loss_spec],
        out_shape=[row_shape, loss_shape],
        compiler_params=_PAR,
    )(xf, rprev, q)
    loss_sums.append(ls3)

    codes = jnp.concatenate(idxs, axis=1)
    sums = jnp.stack([jnp.sum(l) for l in loss_sums])[None, :]
    return quants, codes, sums


def kernel(x, C0, C1, C2, C3):
    xf = x.reshape(ROWS, DIM)
    tables = [jnp.pad(C, ((0, 0), (0, DIM))) for C in (C0, C1, C2, C3)]
    cbs = [C[:-1] for C in (C0, C1, C2, C3)]
    cbbs = [c.astype(jnp.bfloat16) for c in cbs]

    quants, codes, sums = _pipeline(xf, *tables, *cbbs, *cbs)

    quants = quants.reshape(x.shape)
    codes = codes.reshape(x.shape[:-1] + (DEPTH,))
    loss = jnp.mean(jnp.sum(sums, axis=0) / (ROWS * DIM))
    return quants, loss, codes
